# baseline (speedup 1.0000x reference)
import functools

import jax
import jax.numpy as jnp
from jax import lax
from jax.experimental import pallas as pl
from jax.experimental.pallas import tpu as pltpu
from jax.experimental.pallas import tpu_sc as plsc

N = 10000
D = 128
H = 64
Z = 32
E1 = 160000
E2 = 160000

NC = 2
NS = 16
NW = NC * NS
K = 128
T1 = 40
EP = NW * T1 * K
NP = 10240
RPT = NP // NS
DW = 16
NB1 = 8
NB2 = 8

_f32 = jnp.float32


def _mesh():
    return plsc.VectorSubcoreMesh(core_axis_name="c", subcore_axis_name="s")


def _ring(tbl, acc, sv, dv, bufs, semg, sems, nch):
    nb = len(bufs)
    lk = nb // 2

    for g in range(lk):
        pltpu.async_copy(tbl.at[sv.at[g]], bufs[g % nb], semg[g % nb])

    def outer(t2, _):
        for b in range(nb):
            t = t2 * nb + b
            pltpu.make_async_copy(tbl.at[sv.at[0]], bufs[b], semg[b]).wait()
            pltpu.async_copy(bufs[b], acc.at[dv.at[t]], sems[b], add=True)
            bg = (b + lk) % nb

            @pl.when(t + lk < nch)
            def _():
                @pl.when(t >= nb - lk)
                def _():
                    pltpu.make_async_copy(bufs[bg], acc.at[dv.at[0]],
                                          sems[bg]).wait()
                pltpu.async_copy(tbl.at[sv.at[t + lk]], bufs[bg], semg[bg])
        return 0
    lax.fori_loop(0, nch // nb, outer, 0)

    for b in range(nb):
        pltpu.make_async_copy(bufs[b], acc.at[dv.at[0]], sems[b]).wait()


def _sc_deg(s1, d1, s2, d2):
    @functools.partial(
        pl.kernel,
        mesh=_mesh(),
        compiler_params=pltpu.CompilerParams(use_tc_tiling_on_sc=False),
        out_type=(
            jax.ShapeDtypeStruct((NC, NP, DW), _f32),
            jax.ShapeDtypeStruct((NC, NP, DW), _f32),
        ),
        scratch_types=[
            pltpu.VMEM((T1, K), jnp.int32),
            pltpu.VMEM((T1, K), jnp.int32),
            pltpu.VMEM((K, DW), _f32),
            pltpu.VMEM_SHARED((NP, DW), _f32),
            pltpu.VMEM_SHARED((NP, DW), _f32),
        ],
    )
    def k(s1h, d1h, s2h, d2h, dego_o, degi_o, sv, dv, onesv, degosh, degish):
        c = lax.axis_index("c")
        s = lax.axis_index("s")
        w = c * NS + s
        r0 = s * RPT

        def z16(i, _):
            onesv[i, pl.ds(0, 16)] = jnp.zeros((16,), _f32)
            return 0
        lax.fori_loop(0, K, z16, 0)

        for b in range(RPT // K):
            pltpu.sync_copy(onesv, degosh.at[pl.ds(r0 + b * K, K), :])
            pltpu.sync_copy(onesv, degish.at[pl.ds(r0 + b * K, K), :])

        def s16(i, _):
            onesv[i, pl.ds(0, 16)] = jnp.ones((16,), _f32)
            return 0
        lax.fori_loop(0, K, s16, 0)

        plsc.subcore_barrier()

        for sh, dh in ((s1h, d1h), (s2h, d2h)):
            pltpu.sync_copy(sh.at[pl.ds(w * T1, T1), :], sv)
            pltpu.sync_copy(dh.at[pl.ds(w * T1, T1), :], dv)

            def step(t, _):
                pltpu.sync_copy(onesv, degosh.at[sv.at[t]], add=True)
                pltpu.sync_copy(onesv, degish.at[dv.at[t]], add=True)
                return 0
            lax.fori_loop(0, T1, step, 0)

        plsc.subcore_barrier()
        pltpu.sync_copy(degosh.at[pl.ds(r0, RPT), :], dego_o.at[c, pl.ds(r0, RPT), :])
        pltpu.sync_copy(degish.at[pl.ds(r0, RPT), :], degi_o.at[c, pl.ds(r0, RPT), :])

    return k(s1, d1, s2, d2)


def _sc_agg64(r1lo, r1hi, r2lo, r2hi, s1, d1, s2, d2, dep):
    TT = 2 * T1

    @functools.partial(
        pl.kernel,
        mesh=_mesh(),
        compiler_params=pltpu.CompilerParams(use_tc_tiling_on_sc=False),
        out_type=jax.ShapeDtypeStruct((2, NP, H), _f32),
        scratch_types=(
            [pltpu.VMEM((TT, K), jnp.int32),
             pltpu.VMEM((TT, K), jnp.int32)]
            + [pltpu.VMEM((K, H), _f32)] * NB1
            + [pltpu.VMEM_SHARED((NP, H), _f32)]
            + [pltpu.SemaphoreType.DMA] * (2 * NB1)
        ),
    )
    def k(r1lo_h, r1hi_h, r2lo_h, r2hi_h, s1h, d1h, s2h, d2h, dep_h,
          agg_o, sv, dv, *scr):
        del dep_h
        bufs = scr[:NB1]
        aggsh = scr[NB1]
        semg = scr[NB1 + 1:NB1 + 1 + NB1]
        sems = scr[NB1 + 1 + NB1:]
        c = lax.axis_index("c")
        s = lax.axis_index("s")
        r0 = s * RPT

        def z64(i, _):
            bufs[0][i // 4, pl.ds((i % 4) * 16, 16)] = jnp.zeros((16,), _f32)
            return 0
        lax.fori_loop(0, K * (H // 16), z64, 0)

        for b in range(RPT // K):
            pltpu.sync_copy(bufs[0], aggsh.at[pl.ds(r0 + b * K, K), :])

        plsc.subcore_barrier()

        for half, tables in ((0, (r1lo_h, r2lo_h)), (1, (r1hi_h, r2hi_h))):
            @pl.when(c == half)
            def _():
                for tbl, sh, dh in ((tables[0], s1h, d1h),
                                    (tables[1], s2h, d2h)):
                    pltpu.sync_copy(sh.at[pl.ds(s * TT, TT), :], sv)
                    pltpu.sync_copy(dh.at[pl.ds(s * TT, TT), :], dv)
                    _ring(tbl, aggsh, sv, dv, bufs, semg, sems, TT)

        plsc.subcore_barrier()
        pltpu.sync_copy(aggsh.at[pl.ds(r0, RPT), :], agg_o.at[c, pl.ds(r0, RPT), :])

    return k(r1lo, r1hi, r2lo, r2hi, s1, d1, s2, d2, dep)


def _sc_phase2(hn, sall, dall):
    T2 = 2 * T1

    @functools.partial(
        pl.kernel,
        mesh=_mesh(),
        compiler_params=pltpu.CompilerParams(use_tc_tiling_on_sc=False),
        out_type=jax.ShapeDtypeStruct((NC, NP, H), _f32),
        scratch_types=(
            [pltpu.VMEM((T2, K), jnp.int32),
             pltpu.VMEM((T2, K), jnp.int32)]
            + [pltpu.VMEM((K, H), _f32)] * NB2
            + [pltpu.VMEM_SHARED((NP, H), _f32)]
            + [pltpu.SemaphoreType.DMA] * (2 * NB2)
        ),
    )
    def k(hnh, sh, dh, a_o, sv, dv, *scr):
        bufs = scr[:NB2]
        accsh = scr[NB2]
        semg = scr[NB2 + 1:NB2 + 1 + NB2]
        sems = scr[NB2 + 1 + NB2:]
        c = lax.axis_index("c")
        s = lax.axis_index("s")
        w = c * NS + s
        r0 = s * RPT

        def z64(i, _):
            bufs[0][i // 4, pl.ds((i % 4) * 16, 16)] = jnp.zeros((16,), _f32)
            return 0
        lax.fori_loop(0, K * (H // 16), z64, 0)

        for b in range(RPT // K):
            pltpu.sync_copy(bufs[0], accsh.at[pl.ds(r0 + b * K, K), :])

        plsc.subcore_barrier()

        pltpu.sync_copy(sh.at[pl.ds(w * T2, T2), :], sv)
        pltpu.sync_copy(dh.at[pl.ds(w * T2, T2), :], dv)
        _ring(hnh, accsh, sv, dv, bufs, semg, sems, T2)

        plsc.subcore_barrier()
        pltpu.sync_copy(accsh.at[pl.ds(r0, RPT), :], a_o.at[c, pl.ds(r0, RPT), :])

    return k(hn, sall, dall)


_BR = 640


def _prep_body(x_ref, r1lo_ref, r1hi_ref, r2lo_ref, r2hi_ref):
    i = pl.program_id(0)
    row = i * _BR + lax.broadcasted_iota(jnp.int32, (_BR, 1), 0)
    mask = row < N
    xv = x_ref[...]
    r1 = jnp.where(mask, jnp.maximum(xv + 1.0, 0.0), 0.0)
    r2 = jnp.where(mask, jnp.maximum(xv + 2.0, 0.0), 0.0)
    r1lo_ref[...] = r1[:, :H]
    r1hi_ref[...] = r1[:, H:]
    r2lo_ref[...] = r2[:, :H]
    r2hi_ref[...] = r2[:, H:]


def _tc_prep(x_pad):
    half = pl.BlockSpec((_BR, H), lambda i: (i, 0))
    return pl.pallas_call(
        _prep_body,
        grid=(NP // _BR,),
        in_specs=[pl.BlockSpec((_BR, D), lambda i: (i, 0))],
        out_specs=(half, half, half, half),
        out_shape=(jax.ShapeDtypeStruct((NP, H), _f32),) * 4,
    )(x_pad)


def _mid_body(x_ref, agg_ref, degop_ref, w1_ref, b1_ref, hn_ref):
    i = pl.program_id(0)
    row = i * _BR + lax.broadcasted_iota(jnp.int32, (_BR, 1), 0)
    xv = x_ref[...]
    xa_lo = xv[:, :H] + agg_ref[0]
    xa_hi = xv[:, H:] + agg_ref[1]
    w1 = w1_ref[...]
    dn = (((1,), (0,)), ((), ()))
    h = jnp.tanh(
        lax.dot_general(xa_lo, w1[:H], dn, precision=lax.Precision.DEFAULT,
                        preferred_element_type=_f32)
        + lax.dot_general(xa_hi, w1[H:], dn, precision=lax.Precision.DEFAULT,
                          preferred_element_type=_f32)
        + b1_ref[...])
    dego = degop_ref[0, :, 0:1] + degop_ref[1, :, 0:1]
    hn = h * lax.rsqrt(jnp.maximum(dego, 1.0))
    hn_ref[...] = jnp.where(row < N, hn, 0.0)


def _tc_mid(x_pad, aggp, degop, W1, b1):
    return pl.pallas_call(
        _mid_body,
        grid=(NP // _BR,),
        in_specs=[
            pl.BlockSpec((_BR, D), lambda i: (i, 0)),
            pl.BlockSpec((2, _BR, H), lambda i: (0, i, 0)),
            pl.BlockSpec((NC, _BR, DW), lambda i: (0, i, 0)),
            pl.BlockSpec((D, H), lambda i: (0, 0)),
            pl.BlockSpec((1, H), lambda i: (0, 0)),
        ],
        out_specs=pl.BlockSpec((_BR, H), lambda i: (i, 0)),
        out_shape=jax.ShapeDtypeStruct((NP, H), _f32),
    )(x_pad, aggp, degop, W1, b1)


_BRO = 400


def _out_body(ap_ref, degip_ref, eps_ref, wm_ref, ws_ref, bm_ref, bs_ref,
              z_ref, m_ref, s_ref):
    degi = degip_ref[0, :, 0:1] + degip_ref[1, :, 0:1]
    a = (ap_ref[0] + ap_ref[1]) * lax.rsqrt(jnp.maximum(degi, 1.0))
    m = lax.dot_general(a, wm_ref[...], (((1,), (0,)), ((), ())),
                        precision=lax.Precision.DEFAULT,
                        preferred_element_type=_f32) + bm_ref[...]
    sd = jnp.maximum(
        lax.dot_general(a, ws_ref[...], (((1,), (0,)), ((), ())),
                        precision=lax.Precision.DEFAULT,
                        preferred_element_type=_f32) + bs_ref[...], 0.0) + 0.0001
    z_ref[...] = eps_ref[...] * sd + m
    m_ref[...] = m
    s_ref[...] = sd


def _tc_out(ap, degip, eps, Wm, Ws, bm, bs):
    return pl.pallas_call(
        _out_body,
        grid=(N // _BRO,),
        in_specs=[
            pl.BlockSpec((NC, _BRO, H), lambda i: (0, i, 0)),
            pl.BlockSpec((NC, _BRO, DW), lambda i: (0, i, 0)),
            pl.BlockSpec((_BRO, Z), lambda i: (i, 0)),
            pl.BlockSpec((H, Z), lambda i: (0, 0)),
            pl.BlockSpec((H, Z), lambda i: (0, 0)),
            pl.BlockSpec((1, Z), lambda i: (0, 0)),
            pl.BlockSpec((1, Z), lambda i: (0, 0)),
        ],
        out_specs=(pl.BlockSpec((_BRO, Z), lambda i: (i, 0)),
                   pl.BlockSpec((_BRO, Z), lambda i: (i, 0)),
                   pl.BlockSpec((_BRO, Z), lambda i: (i, 0))),
        out_shape=(jax.ShapeDtypeStruct((N, Z), _f32),
                   jax.ShapeDtypeStruct((N, Z), _f32),
                   jax.ShapeDtypeStruct((N, Z), _f32)),
    )(ap, degip, eps, Wm, Ws, bm, bs)


def _pad_edges(idx, ep):
    pad = N + jnp.arange(ep - idx.shape[0], dtype=jnp.int32) % (NP - N)
    return jnp.concatenate([idx, pad]).reshape(ep // K, K)


def kernel(x, edges_1, edges_2, eps_noise, W1, b1, Wm, bm, Ws, bs):
    x_pad = jnp.pad(x, ((0, NP - N), (0, 0)))

    s1 = _pad_edges(edges_1[0], EP)
    d1 = _pad_edges(edges_1[1], EP)
    s2 = _pad_edges(edges_2[0], EP)
    d2 = _pad_edges(edges_2[1], EP)

    degop, degip = _sc_deg(s1, d1, s2, d2)
    r1lo, r1hi, r2lo, r2hi = _tc_prep(x_pad)
    aggp = _sc_agg64(r1lo, r1hi, r2lo, r2hi, s1, d1, s2, d2, degop)

    hn = _tc_mid(x_pad, aggp, degop, W1, b1.reshape(1, H))

    sall = jnp.concatenate([s1, s2], axis=0)
    dall = jnp.concatenate([d1, d2], axis=0)
    ap = _sc_phase2(hn, sall, dall)

    z, m, sd = _tc_out(ap, degip, eps_noise, Wm, Ws,
                       bm.reshape(1, Z), bs.reshape(1, Z))
    return (z, m, sd)


# speedup vs baseline: 1.0358x; 1.0358x over previous
import functools

import jax
import jax.numpy as jnp
from jax import lax
from jax.experimental import pallas as pl
from jax.experimental.pallas import tpu as pltpu
from jax.experimental.pallas import tpu_sc as plsc

N = 10000
D = 128
H = 64
Z = 32
E1 = 160000
E2 = 160000

NC = 2
NS = 16
NW = NC * NS
K = 128
T1 = 40
EP = NW * T1 * K
NP = 10240
RPT = NP // NS
DW = 16
NB1 = 8
NB2 = 8

_f32 = jnp.float32


def _mesh():
    return plsc.VectorSubcoreMesh(core_axis_name="c", subcore_axis_name="s")


def _ring(tbl, acc, sv, dv, bufs, semg, sems, nch):
    nb = len(bufs)
    lk = nb - 2

    for g in range(lk):
        pltpu.async_copy(tbl.at[sv.at[g]], bufs[g % nb], semg[g % nb])

    def outer(t2, _):
        for b in range(nb):
            t = t2 * nb + b
            pltpu.make_async_copy(tbl.at[sv.at[0]], bufs[b], semg[b]).wait()
            pltpu.async_copy(bufs[b], acc.at[dv.at[t]], sems[b], add=True)
            bg = (b + lk) % nb

            @pl.when(t + lk < nch)
            def _():
                @pl.when(t >= nb - lk)
                def _():
                    pltpu.make_async_copy(bufs[bg], acc.at[dv.at[0]],
                                          sems[bg]).wait()
                pltpu.async_copy(tbl.at[sv.at[t + lk]], bufs[bg], semg[bg])
        return 0
    lax.fori_loop(0, nch // nb, outer, 0)

    for b in range(nb):
        pltpu.make_async_copy(bufs[b], acc.at[dv.at[0]], sems[b]).wait()


def _sc_deg(s1, d1, s2, d2):
    @functools.partial(
        pl.kernel,
        mesh=_mesh(),
        compiler_params=pltpu.CompilerParams(use_tc_tiling_on_sc=False),
        out_type=(
            jax.ShapeDtypeStruct((NC, NP, DW), _f32),
            jax.ShapeDtypeStruct((NC, NP, DW), _f32),
        ),
        scratch_types=[
            pltpu.VMEM((T1, K), jnp.int32),
            pltpu.VMEM((T1, K), jnp.int32),
            pltpu.VMEM((K, DW), _f32),
            pltpu.VMEM_SHARED((NP, DW), _f32),
            pltpu.VMEM_SHARED((NP, DW), _f32),
            pltpu.SemaphoreType.DMA,
            pltpu.SemaphoreType.DMA,
        ],
    )
    def k(s1h, d1h, s2h, d2h, dego_o, degi_o, sv, dv, onesv, degosh, degish,
          semo, semi):
        c = lax.axis_index("c")
        s = lax.axis_index("s")
        w = c * NS + s
        r0 = s * RPT

        def z16(i, _):
            onesv[i, pl.ds(0, 16)] = jnp.zeros((16,), _f32)
            return 0
        lax.fori_loop(0, K, z16, 0)

        for b in range(RPT // K):
            pltpu.sync_copy(onesv, degosh.at[pl.ds(r0 + b * K, K), :])
            pltpu.sync_copy(onesv, degish.at[pl.ds(r0 + b * K, K), :])

        def s16(i, _):
            onesv[i, pl.ds(0, 16)] = jnp.ones((16,), _f32)
            return 0
        lax.fori_loop(0, K, s16, 0)

        plsc.subcore_barrier()

        G = 8
        for sh, dh in ((s1h, d1h), (s2h, d2h)):
            pltpu.sync_copy(sh.at[pl.ds(w * T1, T1), :], sv)
            pltpu.sync_copy(dh.at[pl.ds(w * T1, T1), :], dv)

            def grp(g, _):
                for j in range(G):
                    t = g * G + j
                    pltpu.async_copy(onesv, degosh.at[sv.at[t]], semo, add=True)
                    pltpu.async_copy(onesv, degish.at[dv.at[t]], semi, add=True)

                @pl.when(g > 0)
                def _():
                    for j in range(G):
                        pltpu.make_async_copy(onesv, degosh.at[sv.at[0]],
                                              semo).wait()
                        pltpu.make_async_copy(onesv, degish.at[dv.at[0]],
                                              semi).wait()
                return 0
            lax.fori_loop(0, T1 // G, grp, 0)
            for j in range(G):
                pltpu.make_async_copy(onesv, degosh.at[sv.at[0]], semo).wait()
                pltpu.make_async_copy(onesv, degish.at[dv.at[0]], semi).wait()

        plsc.subcore_barrier()
        pltpu.sync_copy(degosh.at[pl.ds(r0, RPT), :], dego_o.at[c, pl.ds(r0, RPT), :])
        pltpu.sync_copy(degish.at[pl.ds(r0, RPT), :], degi_o.at[c, pl.ds(r0, RPT), :])

    return k(s1, d1, s2, d2)


def _sc_agg64(r1lo, r1hi, r2lo, r2hi, s1, d1, s2, d2, dep):
    TT = 2 * T1

    @functools.partial(
        pl.kernel,
        mesh=_mesh(),
        compiler_params=pltpu.CompilerParams(use_tc_tiling_on_sc=False),
        out_type=jax.ShapeDtypeStruct((2, NP, H), _f32),
        scratch_types=(
            [pltpu.VMEM((TT, K), jnp.int32),
             pltpu.VMEM((TT, K), jnp.int32)]
            + [pltpu.VMEM((K, H), _f32)] * NB1
            + [pltpu.VMEM_SHARED((NP, H), _f32)]
            + [pltpu.SemaphoreType.DMA] * (2 * NB1)
        ),
    )
    def k(r1lo_h, r1hi_h, r2lo_h, r2hi_h, s1h, d1h, s2h, d2h, dep_h,
          agg_o, sv, dv, *scr):
        del dep_h
        bufs = scr[:NB1]
        aggsh = scr[NB1]
        semg = scr[NB1 + 1:NB1 + 1 + NB1]
        sems = scr[NB1 + 1 + NB1:]
        c = lax.axis_index("c")
        s = lax.axis_index("s")
        r0 = s * RPT

        def z64(i, _):
            bufs[0][i // 4, pl.ds((i % 4) * 16, 16)] = jnp.zeros((16,), _f32)
            return 0
        lax.fori_loop(0, K * (H // 16), z64, 0)

        for b in range(RPT // K):
            pltpu.sync_copy(bufs[0], aggsh.at[pl.ds(r0 + b * K, K), :])

        plsc.subcore_barrier()

        for half, tables in ((0, (r1lo_h, r2lo_h)), (1, (r1hi_h, r2hi_h))):
            @pl.when(c == half)
            def _():
                for tbl, sh, dh in ((tables[0], s1h, d1h),
                                    (tables[1], s2h, d2h)):
                    pltpu.sync_copy(sh.at[pl.ds(s * TT, TT), :], sv)
                    pltpu.sync_copy(dh.at[pl.ds(s * TT, TT), :], dv)
                    _ring(tbl, aggsh, sv, dv, bufs, semg, sems, TT)

        plsc.subcore_barrier()
        pltpu.sync_copy(aggsh.at[pl.ds(r0, RPT), :], agg_o.at[c, pl.ds(r0, RPT), :])

    return k(r1lo, r1hi, r2lo, r2hi, s1, d1, s2, d2, dep)


def _sc_phase2(hn, s1, d1, s2, d2):
    @functools.partial(
        pl.kernel,
        mesh=_mesh(),
        compiler_params=pltpu.CompilerParams(use_tc_tiling_on_sc=False),
        out_type=jax.ShapeDtypeStruct((NC, NP, H), _f32),
        scratch_types=(
            [pltpu.VMEM((T1, K), jnp.int32),
             pltpu.VMEM((T1, K), jnp.int32)]
            + [pltpu.VMEM((K, H), _f32)] * NB2
            + [pltpu.VMEM_SHARED((NP, H), _f32)]
            + [pltpu.SemaphoreType.DMA] * (2 * NB2)
        ),
    )
    def k(hnh, s1h, d1h, s2h, d2h, a_o, sv, dv, *scr):
        bufs = scr[:NB2]
        accsh = scr[NB2]
        semg = scr[NB2 + 1:NB2 + 1 + NB2]
        sems = scr[NB2 + 1 + NB2:]
        c = lax.axis_index("c")
        s = lax.axis_index("s")
        w = c * NS + s
        r0 = s * RPT

        def z64(i, _):
            bufs[0][i // 4, pl.ds((i % 4) * 16, 16)] = jnp.zeros((16,), _f32)
            return 0
        lax.fori_loop(0, K * (H // 16), z64, 0)

        for b in range(RPT // K):
            pltpu.sync_copy(bufs[0], accsh.at[pl.ds(r0 + b * K, K), :])

        plsc.subcore_barrier()

        for sh, dh in ((s1h, d1h), (s2h, d2h)):
            pltpu.sync_copy(sh.at[pl.ds(w * T1, T1), :], sv)
            pltpu.sync_copy(dh.at[pl.ds(w * T1, T1), :], dv)
            _ring(hnh, accsh, sv, dv, bufs, semg, sems, T1)

        plsc.subcore_barrier()
        pltpu.sync_copy(accsh.at[pl.ds(r0, RPT), :], a_o.at[c, pl.ds(r0, RPT), :])

    return k(hn, s1, d1, s2, d2)


_BR = 640


def _prep_body(x_ref, r1lo_ref, r1hi_ref, r2lo_ref, r2hi_ref):
    i = pl.program_id(0)
    row = i * _BR + lax.broadcasted_iota(jnp.int32, (_BR, 1), 0)
    mask = row < N
    xv = x_ref[...]
    r1 = jnp.where(mask, jnp.maximum(xv + 1.0, 0.0), 0.0)
    r2 = jnp.where(mask, jnp.maximum(xv + 2.0, 0.0), 0.0)
    r1lo_ref[...] = r1[:, :H]
    r1hi_ref[...] = r1[:, H:]
    r2lo_ref[...] = r2[:, :H]
    r2hi_ref[...] = r2[:, H:]


def _tc_prep(x_pad):
    half = pl.BlockSpec((_BR, H), lambda i: (i, 0))
    return pl.pallas_call(
        _prep_body,
        grid=(NP // _BR,),
        in_specs=[pl.BlockSpec((_BR, D), lambda i: (i, 0))],
        out_specs=(half, half, half, half),
        out_shape=(jax.ShapeDtypeStruct((NP, H), _f32),) * 4,
    )(x_pad)


def _mid_body(x_ref, agg_ref, degop_ref, w1_ref, b1_ref, hn_ref):
    i = pl.program_id(0)
    row = i * _BR + lax.broadcasted_iota(jnp.int32, (_BR, 1), 0)
    xv = x_ref[...]
    xa_lo = xv[:, :H] + agg_ref[0]
    xa_hi = xv[:, H:] + agg_ref[1]
    w1 = w1_ref[...]
    dn = (((1,), (0,)), ((), ()))
    h = jnp.tanh(
        lax.dot_general(xa_lo, w1[:H], dn, precision=lax.Precision.DEFAULT,
                        preferred_element_type=_f32)
        + lax.dot_general(xa_hi, w1[H:], dn, precision=lax.Precision.DEFAULT,
                          preferred_element_type=_f32)
        + b1_ref[...])
    dego = degop_ref[0, :, 0:1] + degop_ref[1, :, 0:1]
    hn = h * lax.rsqrt(jnp.maximum(dego, 1.0))
    hn_ref[...] = jnp.where(row < N, hn, 0.0)


def _tc_mid(x_pad, aggp, degop, W1, b1):
    return pl.pallas_call(
        _mid_body,
        grid=(NP // _BR,),
        in_specs=[
            pl.BlockSpec((_BR, D), lambda i: (i, 0)),
            pl.BlockSpec((2, _BR, H), lambda i: (0, i, 0)),
            pl.BlockSpec((NC, _BR, DW), lambda i: (0, i, 0)),
            pl.BlockSpec((D, H), lambda i: (0, 0)),
            pl.BlockSpec((1, H), lambda i: (0, 0)),
        ],
        out_specs=pl.BlockSpec((_BR, H), lambda i: (i, 0)),
        out_shape=jax.ShapeDtypeStruct((NP, H), _f32),
    )(x_pad, aggp, degop, W1, b1)


_BRO = 400


def _out_body(ap_ref, degip_ref, eps_ref, wm_ref, ws_ref, bm_ref, bs_ref,
              z_ref, m_ref, s_ref):
    degi = degip_ref[0, :, 0:1] + degip_ref[1, :, 0:1]
    a = (ap_ref[0] + ap_ref[1]) * lax.rsqrt(jnp.maximum(degi, 1.0))
    m = lax.dot_general(a, wm_ref[...], (((1,), (0,)), ((), ())),
                        precision=lax.Precision.DEFAULT,
                        preferred_element_type=_f32) + bm_ref[...]
    sd = jnp.maximum(
        lax.dot_general(a, ws_ref[...], (((1,), (0,)), ((), ())),
                        precision=lax.Precision.DEFAULT,
                        preferred_element_type=_f32) + bs_ref[...], 0.0) + 0.0001
    z_ref[...] = eps_ref[...] * sd + m
    m_ref[...] = m
    s_ref[...] = sd


def _tc_out(ap, degip, eps, Wm, Ws, bm, bs):
    return pl.pallas_call(
        _out_body,
        grid=(N // _BRO,),
        in_specs=[
            pl.BlockSpec((NC, _BRO, H), lambda i: (0, i, 0)),
            pl.BlockSpec((NC, _BRO, DW), lambda i: (0, i, 0)),
            pl.BlockSpec((_BRO, Z), lambda i: (i, 0)),
            pl.BlockSpec((H, Z), lambda i: (0, 0)),
            pl.BlockSpec((H, Z), lambda i: (0, 0)),
            pl.BlockSpec((1, Z), lambda i: (0, 0)),
            pl.BlockSpec((1, Z), lambda i: (0, 0)),
        ],
        out_specs=(pl.BlockSpec((_BRO, Z), lambda i: (i, 0)),
                   pl.BlockSpec((_BRO, Z), lambda i: (i, 0)),
                   pl.BlockSpec((_BRO, Z), lambda i: (i, 0))),
        out_shape=(jax.ShapeDtypeStruct((N, Z), _f32),
                   jax.ShapeDtypeStruct((N, Z), _f32),
                   jax.ShapeDtypeStruct((N, Z), _f32)),
    )(ap, degip, eps, Wm, Ws, bm, bs)


def _pad_edges(idx, ep):
    pad = N + jnp.arange(ep - idx.shape[0], dtype=jnp.int32) % (NP - N)
    return jnp.concatenate([idx, pad]).reshape(ep // K, K)


def kernel(x, edges_1, edges_2, eps_noise, W1, b1, Wm, bm, Ws, bs):
    x_pad = jnp.pad(x, ((0, NP - N), (0, 0)))

    s1 = _pad_edges(edges_1[0], EP)
    d1 = _pad_edges(edges_1[1], EP)
    s2 = _pad_edges(edges_2[0], EP)
    d2 = _pad_edges(edges_2[1], EP)

    degop, degip = _sc_deg(s1, d1, s2, d2)
    r1lo, r1hi, r2lo, r2hi = _tc_prep(x_pad)
    aggp = _sc_agg64(r1lo, r1hi, r2lo, r2hi, s1, d1, s2, d2, degop)

    hn = _tc_mid(x_pad, aggp, degop, W1, b1.reshape(1, H))

    ap = _sc_phase2(hn, s1, d1, s2, d2)

    z, m, sd = _tc_out(ap, degip, eps_noise, Wm, Ws,
                       bm.reshape(1, Z), bs.reshape(1, Z))
    return (z, m, sd)


# speedup vs baseline: 1.0946x; 1.0567x over previous
import functools

import jax
import jax.numpy as jnp
from jax import lax
from jax.experimental import pallas as pl
from jax.experimental.pallas import tpu as pltpu
from jax.experimental.pallas import tpu_sc as plsc

N = 10000
D = 128
H = 64
Z = 32
E1 = 160000
E2 = 160000

NC = 2
NS = 16
NW = NC * NS
K = 128
T1 = 40
EP = NW * T1 * K
NP = 10240
RPT = NP // NS
DW = 16
NB1 = 8
NB2 = 8

_f32 = jnp.float32


def _mesh():
    return plsc.VectorSubcoreMesh(core_axis_name="c", subcore_axis_name="s")


def _ring(tbl, acc, sv, dv, bufs, semg, sems, nch):
    nb = len(bufs)
    lk = nb - 2

    for g in range(lk):
        pltpu.async_copy(tbl.at[sv.at[g]], bufs[g % nb], semg[g % nb])

    def outer(t2, _):
        for b in range(nb):
            t = t2 * nb + b
            pltpu.make_async_copy(tbl.at[sv.at[0]], bufs[b], semg[b]).wait()
            pltpu.async_copy(bufs[b], acc.at[dv.at[t]], sems[b], add=True)
            bg = (b + lk) % nb

            @pl.when(t + lk < nch)
            def _():
                @pl.when(t >= nb - lk)
                def _():
                    pltpu.make_async_copy(bufs[bg], acc.at[dv.at[0]],
                                          sems[bg]).wait()
                pltpu.async_copy(tbl.at[sv.at[t + lk]], bufs[bg], semg[bg])
        return 0
    lax.fori_loop(0, nch // nb, outer, 0)

    for b in range(nb):
        pltpu.make_async_copy(bufs[b], acc.at[dv.at[0]], sems[b]).wait()


def _sc_deg(s1, d1, s2, d2):
    @functools.partial(
        pl.kernel,
        mesh=_mesh(),
        compiler_params=pltpu.CompilerParams(use_tc_tiling_on_sc=False),
        out_type=(
            jax.ShapeDtypeStruct((NC, NP, DW), _f32),
            jax.ShapeDtypeStruct((NC, NP, DW), _f32),
        ),
        scratch_types=[
            pltpu.VMEM((T1, K), jnp.int32),
            pltpu.VMEM((T1, K), jnp.int32),
            pltpu.VMEM((K, DW), _f32),
            pltpu.VMEM_SHARED((NP, DW), _f32),
            pltpu.VMEM_SHARED((NP, DW), _f32),
            pltpu.SemaphoreType.DMA,
            pltpu.SemaphoreType.DMA,
        ],
    )
    def k(s1h, d1h, s2h, d2h, dego_o, degi_o, sv, dv, onesv, degosh, degish,
          semo, semi):
        c = lax.axis_index("c")
        s = lax.axis_index("s")
        w = c * NS + s
        r0 = s * RPT

        def z16(i, _):
            onesv[i, pl.ds(0, 16)] = jnp.zeros((16,), _f32)
            return 0
        lax.fori_loop(0, K, z16, 0)

        for b in range(RPT // K):
            pltpu.sync_copy(onesv, degosh.at[pl.ds(r0 + b * K, K), :])
            pltpu.sync_copy(onesv, degish.at[pl.ds(r0 + b * K, K), :])

        def s16(i, _):
            onesv[i, pl.ds(0, 16)] = jnp.ones((16,), _f32)
            return 0
        lax.fori_loop(0, K, s16, 0)

        plsc.subcore_barrier()

        G = 8
        for sh, dh in ((s1h, d1h), (s2h, d2h)):
            pltpu.sync_copy(sh.at[pl.ds(w * T1, T1), :], sv)
            pltpu.sync_copy(dh.at[pl.ds(w * T1, T1), :], dv)

            def grp(g, _):
                for j in range(G):
                    t = g * G + j
                    pltpu.async_copy(onesv, degosh.at[sv.at[t]], semo, add=True)
                    pltpu.async_copy(onesv, degish.at[dv.at[t]], semi, add=True)

                @pl.when(g > 0)
                def _():
                    for j in range(G):
                        pltpu.make_async_copy(onesv, degosh.at[sv.at[0]],
                                              semo).wait()
                        pltpu.make_async_copy(onesv, degish.at[dv.at[0]],
                                              semi).wait()
                return 0
            lax.fori_loop(0, T1 // G, grp, 0)
            for j in range(G):
                pltpu.make_async_copy(onesv, degosh.at[sv.at[0]], semo).wait()
                pltpu.make_async_copy(onesv, degish.at[dv.at[0]], semi).wait()

        plsc.subcore_barrier()
        pltpu.sync_copy(degosh.at[pl.ds(r0, RPT), :], dego_o.at[c, pl.ds(r0, RPT), :])
        pltpu.sync_copy(degish.at[pl.ds(r0, RPT), :], degi_o.at[c, pl.ds(r0, RPT), :])

    return k(s1, d1, s2, d2)


def _sc_agg64(r1lo, r1hi, r2lo, r2hi, s1, d1, s2, d2, dep):
    TT = 2 * T1

    @functools.partial(
        pl.kernel,
        mesh=_mesh(),
        compiler_params=pltpu.CompilerParams(use_tc_tiling_on_sc=False),
        out_type=jax.ShapeDtypeStruct((2, NP, H), _f32),
        scratch_types=(
            [pltpu.VMEM((TT, K), jnp.int32),
             pltpu.VMEM((TT, K), jnp.int32)]
            + [pltpu.VMEM((K, H), _f32)] * NB1
            + [pltpu.VMEM_SHARED((NP, H), _f32)]
            + [pltpu.SemaphoreType.DMA] * (2 * NB1)
        ),
    )
    def k(r1lo_h, r1hi_h, r2lo_h, r2hi_h, s1h, d1h, s2h, d2h, dep_h,
          agg_o, sv, dv, *scr):
        del dep_h
        bufs = scr[:NB1]
        aggsh = scr[NB1]
        semg = scr[NB1 + 1:NB1 + 1 + NB1]
        sems = scr[NB1 + 1 + NB1:]
        c = lax.axis_index("c")
        s = lax.axis_index("s")
        r0 = s * RPT

        def z64(i, _):
            bufs[0][i // 4, pl.ds((i % 4) * 16, 16)] = jnp.zeros((16,), _f32)
            return 0
        lax.fori_loop(0, K * (H // 16), z64, 0)

        for b in range(RPT // K):
            pltpu.sync_copy(bufs[0], aggsh.at[pl.ds(r0 + b * K, K), :])

        plsc.subcore_barrier()

        for half, tables in ((0, (r1lo_h, r2lo_h)), (1, (r1hi_h, r2hi_h))):
            @pl.when(c == half)
            def _():
                for tbl, sh, dh in ((tables[0], s1h, d1h),
                                    (tables[1], s2h, d2h)):
                    pltpu.sync_copy(sh.at[pl.ds(s * TT, TT), :], sv)
                    pltpu.sync_copy(dh.at[pl.ds(s * TT, TT), :], dv)
                    _ring(tbl, aggsh, sv, dv, bufs, semg, sems, TT)

        plsc.subcore_barrier()
        pltpu.sync_copy(aggsh.at[pl.ds(r0, RPT), :], agg_o.at[c, pl.ds(r0, RPT), :])

    return k(r1lo, r1hi, r2lo, r2hi, s1, d1, s2, d2, dep)


def _sc_phase2(hn, s1, d1, s2, d2):
    @functools.partial(
        pl.kernel,
        mesh=_mesh(),
        compiler_params=pltpu.CompilerParams(use_tc_tiling_on_sc=False),
        out_type=jax.ShapeDtypeStruct((NC, NP, H), _f32),
        scratch_types=(
            [pltpu.VMEM((T1, K), jnp.int32),
             pltpu.VMEM((T1, K), jnp.int32)]
            + [pltpu.VMEM((K, H), _f32)] * NB2
            + [pltpu.VMEM_SHARED((NP, H), _f32)]
            + [pltpu.SemaphoreType.DMA] * (2 * NB2)
        ),
    )
    def k(hnh, s1h, d1h, s2h, d2h, a_o, sv, dv, *scr):
        bufs = scr[:NB2]
        accsh = scr[NB2]
        semg = scr[NB2 + 1:NB2 + 1 + NB2]
        sems = scr[NB2 + 1 + NB2:]
        c = lax.axis_index("c")
        s = lax.axis_index("s")
        w = c * NS + s
        r0 = s * RPT

        def z64(i, _):
            bufs[0][i // 4, pl.ds((i % 4) * 16, 16)] = jnp.zeros((16,), _f32)
            return 0
        lax.fori_loop(0, K * (H // 16), z64, 0)

        for b in range(RPT // K):
            pltpu.sync_copy(bufs[0], accsh.at[pl.ds(r0 + b * K, K), :])

        plsc.subcore_barrier()

        for sh, dh in ((s1h, d1h), (s2h, d2h)):
            pltpu.sync_copy(sh.at[pl.ds(w * T1, T1), :], sv)
            pltpu.sync_copy(dh.at[pl.ds(w * T1, T1), :], dv)
            _ring(hnh, accsh, sv, dv, bufs, semg, sems, T1)

        plsc.subcore_barrier()
        pltpu.sync_copy(accsh.at[pl.ds(r0, RPT), :], a_o.at[c, pl.ds(r0, RPT), :])

    return k(hn, s1, d1, s2, d2)


_BR = 1280


def _prep_body(x_ref, r1lo_ref, r1hi_ref, r2lo_ref, r2hi_ref):
    i = pl.program_id(0)
    row = i * _BR + lax.broadcasted_iota(jnp.int32, (_BR, 1), 0)
    mask = row < N
    xv = x_ref[...]
    r1 = jnp.where(mask, jnp.maximum(xv + 1.0, 0.0), 0.0)
    r2 = jnp.where(mask, jnp.maximum(xv + 2.0, 0.0), 0.0)
    r1lo_ref[...] = r1[:, :H]
    r1hi_ref[...] = r1[:, H:]
    r2lo_ref[...] = r2[:, :H]
    r2hi_ref[...] = r2[:, H:]


def _tc_prep(x_pad):
    half = pl.BlockSpec((_BR, H), lambda i: (i, 0))
    return pl.pallas_call(
        _prep_body,
        grid=(NP // _BR,),
        in_specs=[pl.BlockSpec((_BR, D), lambda i: (i, 0))],
        out_specs=(half, half, half, half),
        out_shape=(jax.ShapeDtypeStruct((NP, H), _f32),) * 4,
    )(x_pad)


def _mid_body(x_ref, agg_ref, degop_ref, w1_ref, b1_ref, hn_ref):
    i = pl.program_id(0)
    row = i * _BR + lax.broadcasted_iota(jnp.int32, (_BR, 1), 0)
    xv = x_ref[...]
    xa_lo = xv[:, :H] + agg_ref[0]
    xa_hi = xv[:, H:] + agg_ref[1]
    w1 = w1_ref[...]
    dn = (((1,), (0,)), ((), ()))
    h = jnp.tanh(
        lax.dot_general(xa_lo, w1[:H], dn, precision=lax.Precision.DEFAULT,
                        preferred_element_type=_f32)
        + lax.dot_general(xa_hi, w1[H:], dn, precision=lax.Precision.DEFAULT,
                          preferred_element_type=_f32)
        + b1_ref[...])
    dego = degop_ref[0, :, 0:1] + degop_ref[1, :, 0:1]
    hn = h * lax.rsqrt(jnp.maximum(dego, 1.0))
    hn_ref[...] = jnp.where(row < N, hn, 0.0)


def _tc_mid(x_pad, aggp, degop, W1, b1):
    return pl.pallas_call(
        _mid_body,
        grid=(NP // _BR,),
        in_specs=[
            pl.BlockSpec((_BR, D), lambda i: (i, 0)),
            pl.BlockSpec((2, _BR, H), lambda i: (0, i, 0)),
            pl.BlockSpec((NC, _BR, DW), lambda i: (0, i, 0)),
            pl.BlockSpec((D, H), lambda i: (0, 0)),
            pl.BlockSpec((1, H), lambda i: (0, 0)),
        ],
        out_specs=pl.BlockSpec((_BR, H), lambda i: (i, 0)),
        out_shape=jax.ShapeDtypeStruct((NP, H), _f32),
    )(x_pad, aggp, degop, W1, b1)


_BRO = 1000


def _out_body(ap_ref, degip_ref, eps_ref, wm_ref, ws_ref, bm_ref, bs_ref,
              z_ref, m_ref, s_ref):
    degi = degip_ref[0, :, 0:1] + degip_ref[1, :, 0:1]
    a = (ap_ref[0] + ap_ref[1]) * lax.rsqrt(jnp.maximum(degi, 1.0))
    m = lax.dot_general(a, wm_ref[...], (((1,), (0,)), ((), ())),
                        precision=lax.Precision.DEFAULT,
                        preferred_element_type=_f32) + bm_ref[...]
    sd = jnp.maximum(
        lax.dot_general(a, ws_ref[...], (((1,), (0,)), ((), ())),
                        precision=lax.Precision.DEFAULT,
                        preferred_element_type=_f32) + bs_ref[...], 0.0) + 0.0001
    z_ref[...] = eps_ref[...] * sd + m
    m_ref[...] = m
    s_ref[...] = sd


def _tc_out(ap, degip, eps, Wm, Ws, bm, bs):
    return pl.pallas_call(
        _out_body,
        grid=(N // _BRO,),
        in_specs=[
            pl.BlockSpec((NC, _BRO, H), lambda i: (0, i, 0)),
            pl.BlockSpec((NC, _BRO, DW), lambda i: (0, i, 0)),
            pl.BlockSpec((_BRO, Z), lambda i: (i, 0)),
            pl.BlockSpec((H, Z), lambda i: (0, 0)),
            pl.BlockSpec((H, Z), lambda i: (0, 0)),
            pl.BlockSpec((1, Z), lambda i: (0, 0)),
            pl.BlockSpec((1, Z), lambda i: (0, 0)),
        ],
        out_specs=(pl.BlockSpec((_BRO, Z), lambda i: (i, 0)),
                   pl.BlockSpec((_BRO, Z), lambda i: (i, 0)),
                   pl.BlockSpec((_BRO, Z), lambda i: (i, 0))),
        out_shape=(jax.ShapeDtypeStruct((N, Z), _f32),
                   jax.ShapeDtypeStruct((N, Z), _f32),
                   jax.ShapeDtypeStruct((N, Z), _f32)),
    )(ap, degip, eps, Wm, Ws, bm, bs)


def _pad_edges(idx, ep):
    pad = N + jnp.arange(ep - idx.shape[0], dtype=jnp.int32) % (NP - N)
    return jnp.concatenate([idx, pad]).reshape(ep // K, K)


def kernel(x, edges_1, edges_2, eps_noise, W1, b1, Wm, bm, Ws, bs):
    x_pad = jnp.pad(x, ((0, NP - N), (0, 0)))

    s1 = _pad_edges(edges_1[0], EP)
    d1 = _pad_edges(edges_1[1], EP)
    s2 = _pad_edges(edges_2[0], EP)
    d2 = _pad_edges(edges_2[1], EP)

    degop, degip = _sc_deg(s1, d1, s2, d2)
    r1lo, r1hi, r2lo, r2hi = _tc_prep(x_pad)
    aggp = _sc_agg64(r1lo, r1hi, r2lo, r2hi, s1, d1, s2, d2, degop)

    hn = _tc_mid(x_pad, aggp, degop, W1, b1.reshape(1, H))

    ap = _sc_phase2(hn, s1, d1, s2, d2)

    z, m, sd = _tc_out(ap, degip, eps_noise, Wm, Ws,
                       bm.reshape(1, Z), bs.reshape(1, Z))
    return (z, m, sd)
